# baseline (speedup 1.0000x reference)
import functools

import jax
import jax.numpy as jnp
from jax.experimental import pallas as pl
from jax.experimental.pallas import tpu as pltpu
from jax.experimental.pallas import tpu_sc as plsc

B, M, D = 4, 8192, 1024
L = 77
H = 16
NMAX = 2560
RHO_MIN, RHO_MAX = 0.05, 0.5
LAM_T, LAM_M, LAM_S = 1.0, 1.7, 0.05

NTOT = B * NMAX


def _make_sc_gather():
    info = plsc.get_sparse_core_info()
    nw = info.num_cores * info.num_subcores
    nrows = NTOT // 2
    rows_per_w = nrows // nw
    chunk = 40
    nchunk = rows_per_w // chunk
    mesh = plsc.VectorSubcoreMesh(core_axis_name="c", subcore_axis_name="s")

    @functools.partial(
        pl.kernel,
        mesh=mesh,
        out_type=jax.ShapeDtypeStruct((nrows, D), jnp.float32),
        scratch_types=[
            pltpu.VMEM((nchunk, chunk), jnp.int32),
            pltpu.VMEM((chunk, D), jnp.float32),
            pltpu.SemaphoreType.DMA,
        ],
    )
    def gather_k(x_hbm, idx_hbm, out_hbm, idx_v, rows_v, sem):
        wid = jax.lax.axis_index("s") * info.num_cores + jax.lax.axis_index("c")
        base = wid * rows_per_w
        for ci in range(nchunk):
            pltpu.sync_copy(
                idx_hbm.at[pl.ds(base + ci * chunk, chunk)], idx_v.at[ci])
            pltpu.async_copy(x_hbm.at[idx_v.at[ci]], rows_v, sem).wait()
            pltpu.sync_copy(rows_v, out_hbm.at[pl.ds(base + ci * chunk, chunk)])

    return gather_k


_sc_gather = _make_sc_gather()


_TMZ = 1024


def _z_body(x_ref, w_ref, o_ref):
    o_ref[...] = jnp.dot(
        x_ref[...].astype(jnp.bfloat16),
        w_ref[...].astype(jnp.bfloat16),
        preferred_element_type=jnp.float32,
    )


def _z_pallas(rows, W_re):
    n = rows.shape[0]
    return pl.pallas_call(
        _z_body,
        grid=(n // _TMZ,),
        in_specs=[
            pl.BlockSpec((_TMZ, D), lambda i: (i, 0)),
            pl.BlockSpec((D, D), lambda i: (0, 0)),
        ],
        out_specs=pl.BlockSpec((_TMZ, D), lambda i: (i, 0)),
        out_shape=jax.ShapeDtypeStruct((n, D), jnp.float32),
    )(rows, W_re)


def kernel(X_v, Q_t, Wq, Wk, Wv, w_s, a_r, b_r, W_re):
    dh = D // H
    q = (X_v @ Wq).reshape(B, M, H, dh)
    k = (Q_t @ Wk).reshape(B, L, H, dh)
    v = (Q_t @ Wv).reshape(B, L, H, dh)
    logits = jnp.einsum("bmhd,blhd->bhml", q, k) / jnp.sqrt(float(dh))
    attn = jax.nn.softmax(logits, axis=-1)
    ctx = jnp.einsum("bhml,blhd->bmhd", attn, v).reshape(B, M, D)
    s = ctx @ w_s
    pooled = jnp.mean(jax.nn.sigmoid(s), axis=1)
    r = jax.nn.sigmoid(a_r * pooled + b_r)
    rho = RHO_MIN + (RHO_MAX - RHO_MIN) * r
    n_vec = jnp.minimum(jnp.round(rho * M), float(NMAX)).astype(jnp.int32)
    _, top_idx = jax.lax.top_k(s, NMAX)
    flat_idx = (top_idx + (jnp.arange(B, dtype=jnp.int32) * M)[:, None]).reshape(NTOT)
    x_flat = X_v.reshape(B * M, D)
    half = NTOT // 2
    rows0 = _sc_gather(x_flat, flat_idx[:half])
    rows1 = _sc_gather(x_flat, flat_idx[half:])
    Z0 = _z_pallas(rows0, W_re)
    Z1 = _z_pallas(rows1, W_re)
    Z = jnp.concatenate([Z0, Z1], axis=0).reshape(B, NMAX, D)
    M_f = float(M)
    flops_proxy = (rho * M_f) ** 2 / float(NMAX ** 2)
    kv_proxy = rho * M_f / float(NMAX)
    rho_loss = (rho - jnp.mean(rho)) ** 2
    return (
        Z,
        top_idx,
        rho,
        r,
        n_vec,
        jnp.mean(flops_proxy) * LAM_T,
        jnp.mean(kv_proxy) * LAM_M,
        jnp.mean(rho_loss) * LAM_S,
    )


# speedup vs baseline: 1.0760x; 1.0760x over previous
import functools

import jax
import jax.numpy as jnp
from jax.experimental import pallas as pl
from jax.experimental.pallas import tpu as pltpu
from jax.experimental.pallas import tpu_sc as plsc

B, M, D = 4, 8192, 1024
L = 77
H = 16
NMAX = 2560
RHO_MIN, RHO_MAX = 0.05, 0.5
LAM_T, LAM_M, LAM_S = 1.0, 1.7, 0.05

NTOT = B * NMAX


def _make_sc_gather():
    info = plsc.get_sparse_core_info()
    nw = info.num_cores * info.num_subcores
    nrows = NTOT
    rows_per_w = nrows // nw
    chunk = 80
    nchunk = rows_per_w // chunk
    w_per_b = nw // B
    mesh = plsc.VectorSubcoreMesh(core_axis_name="c", subcore_axis_name="s")

    @functools.partial(
        pl.kernel,
        mesh=mesh,
        out_type=jax.ShapeDtypeStruct((nrows, D), jnp.float32),
        scratch_types=[
            pltpu.VMEM((nchunk, chunk), jnp.int32),
            pltpu.VMEM((chunk, D), jnp.float32),
            pltpu.SemaphoreType.DMA,
        ],
    )
    def gather_k(x_hbm, idx_hbm, out_hbm, idx_v, rows_v, sem):
        wid = jax.lax.axis_index("s") * info.num_cores + jax.lax.axis_index("c")
        base = wid * rows_per_w
        row_base = (wid // w_per_b) * M
        for ci in range(nchunk):
            pltpu.sync_copy(
                idx_hbm.at[pl.ds(base + ci * chunk, chunk)], idx_v.at[ci])
            for vi in range(chunk // 16):
                sl = pl.ds(vi * 16, 16)
                idx_v[ci, sl] = idx_v[ci, sl] + row_base
            pltpu.async_copy(x_hbm.at[idx_v.at[ci]], rows_v, sem).wait()
            pltpu.sync_copy(rows_v, out_hbm.at[pl.ds(base + ci * chunk, chunk)])

    return gather_k


_sc_gather = _make_sc_gather()


_TMZ = 1024


def _z_body(x_ref, w_ref, o_ref):
    o_ref[...] = jnp.dot(
        x_ref[...].astype(jnp.bfloat16),
        w_ref[...].astype(jnp.bfloat16),
        preferred_element_type=jnp.float32,
    )


def _z_pallas(rows, W_re):
    n = rows.shape[0]
    return pl.pallas_call(
        _z_body,
        grid=(n // _TMZ,),
        in_specs=[
            pl.BlockSpec((_TMZ, D), lambda i: (i, 0)),
            pl.BlockSpec((D, D), lambda i: (0, 0)),
        ],
        out_specs=pl.BlockSpec((_TMZ, D), lambda i: (i, 0)),
        out_shape=jax.ShapeDtypeStruct((n, D), jnp.float32),
    )(rows, W_re)


def kernel(X_v, Q_t, Wq, Wk, Wv, w_s, a_r, b_r, W_re):
    dh = D // H
    q = (X_v @ Wq).reshape(B, M, H, dh)
    k = (Q_t @ Wk).reshape(B, L, H, dh)
    v = (Q_t @ Wv).reshape(B, L, H, dh)
    logits = jnp.einsum("bmhd,blhd->bhml", q, k) / jnp.sqrt(float(dh))
    attn = jax.nn.softmax(logits, axis=-1)
    ctx = jnp.einsum("bhml,blhd->bmhd", attn, v).reshape(B, M, D)
    s = ctx @ w_s
    pooled = jnp.mean(jax.nn.sigmoid(s), axis=1)
    r = jax.nn.sigmoid(a_r * pooled + b_r)
    rho = RHO_MIN + (RHO_MAX - RHO_MIN) * r
    n_vec = jnp.minimum(jnp.round(rho * M), float(NMAX)).astype(jnp.int32)
    _, top_idx = jax.lax.top_k(s, NMAX)
    rows = _sc_gather(X_v.reshape(B * M, D), top_idx.reshape(NTOT))
    Z = _z_pallas(rows, W_re).reshape(B, NMAX, D)
    M_f = float(M)
    flops_proxy = (rho * M_f) ** 2 / float(NMAX ** 2)
    kv_proxy = rho * M_f / float(NMAX)
    rho_loss = (rho - jnp.mean(rho)) ** 2
    return (
        Z,
        top_idx,
        rho,
        r,
        n_vec,
        jnp.mean(flops_proxy) * LAM_T,
        jnp.mean(kv_proxy) * LAM_M,
        jnp.mean(rho_loss) * LAM_S,
    )


# speedup vs baseline: 1.0838x; 1.0073x over previous
import functools

import jax
import jax.numpy as jnp
from jax.experimental import pallas as pl
from jax.experimental.pallas import tpu as pltpu
from jax.experimental.pallas import tpu_sc as plsc

B, M, D = 4, 8192, 1024
L = 77
H = 16
NMAX = 2560
RHO_MIN, RHO_MAX = 0.05, 0.5
LAM_T, LAM_M, LAM_S = 1.0, 1.7, 0.05

NTOT = B * NMAX


def _make_sc_gather():
    info = plsc.get_sparse_core_info()
    nw = info.num_cores * info.num_subcores
    nrows = NTOT
    rows_per_w = nrows // nw
    chunk = 80
    nchunk = rows_per_w // chunk
    w_per_b = nw // B
    mesh = plsc.VectorSubcoreMesh(core_axis_name="c", subcore_axis_name="s")

    @functools.partial(
        pl.kernel,
        mesh=mesh,
        out_type=jax.ShapeDtypeStruct((nrows, D), jnp.float32),
        scratch_types=[
            pltpu.VMEM((nchunk, chunk), jnp.int32),
            pltpu.VMEM((chunk, D), jnp.float32),
            pltpu.SemaphoreType.DMA,
        ],
    )
    def gather_k(x_hbm, idx_hbm, out_hbm, idx_v, rows_v, sem):
        wid = jax.lax.axis_index("s") * info.num_cores + jax.lax.axis_index("c")
        base = wid * rows_per_w
        row_base = (wid // w_per_b) * M
        for ci in range(nchunk):
            pltpu.sync_copy(
                idx_hbm.at[pl.ds(base + ci * chunk, chunk)], idx_v.at[ci])
            for vi in range(chunk // 16):
                sl = pl.ds(vi * 16, 16)
                idx_v[ci, sl] = idx_v[ci, sl] + row_base
            pltpu.async_copy(x_hbm.at[idx_v.at[ci]], rows_v, sem).wait()
            pltpu.sync_copy(rows_v, out_hbm.at[pl.ds(base + ci * chunk, chunk)])

    return gather_k


_sc_gather = _make_sc_gather()


_TMZ = 1024


def _z_body(x_ref, w_ref, o_ref):
    o_ref[...] = jnp.dot(
        x_ref[...].astype(jnp.bfloat16),
        w_ref[...].astype(jnp.bfloat16),
        preferred_element_type=jnp.float32,
    )


def _z_pallas(rows, W_re):
    n = rows.shape[0]
    return pl.pallas_call(
        _z_body,
        grid=(n // _TMZ,),
        in_specs=[
            pl.BlockSpec((_TMZ, D), lambda i: (i, 0)),
            pl.BlockSpec((D, D), lambda i: (0, 0)),
        ],
        out_specs=pl.BlockSpec((_TMZ, D), lambda i: (i, 0)),
        out_shape=jax.ShapeDtypeStruct((n, D), jnp.float32),
    )(rows, W_re)


def _rho_body(s_ref, a_ref, b_ref, rho_ref, r_ref, nf_ref, aux_ref):
    s = s_ref[...]
    pooled = jnp.mean(jax.nn.sigmoid(s), axis=1, keepdims=True)
    r = jax.nn.sigmoid(a_ref[...] * pooled + b_ref[...])
    rho = RHO_MIN + (RHO_MAX - RHO_MIN) * r
    n_f = jnp.minimum(jnp.round(rho * M), float(NMAX))
    M_f = float(M)
    flops = jnp.mean((rho * M_f) ** 2) / float(NMAX ** 2) * LAM_T
    kv = jnp.mean(rho * M_f) / float(NMAX) * LAM_M
    rl = jnp.mean((rho - jnp.mean(rho)) ** 2) * LAM_S
    rho_ref[...] = rho
    r_ref[...] = r
    nf_ref[...] = n_f
    aux_ref[...] = jnp.concatenate(
        [flops.reshape(1, 1), kv.reshape(1, 1), rl.reshape(1, 1),
         jnp.zeros((1, 1), jnp.float32)], axis=0)


def _rho_pallas(s, a_r, b_r):
    return pl.pallas_call(
        _rho_body,
        out_shape=(
            jax.ShapeDtypeStruct((B, 1), jnp.float32),
            jax.ShapeDtypeStruct((B, 1), jnp.float32),
            jax.ShapeDtypeStruct((B, 1), jnp.float32),
            jax.ShapeDtypeStruct((4, 1), jnp.float32),
        ),
    )(s, a_r.reshape(1, 1), b_r.reshape(1, 1))


def kernel(X_v, Q_t, Wq, Wk, Wv, w_s, a_r, b_r, W_re):
    dh = D // H
    q = (X_v @ Wq).reshape(B, M, H, dh)
    k = (Q_t @ Wk).reshape(B, L, H, dh)
    v = (Q_t @ Wv).reshape(B, L, H, dh)
    logits = jnp.einsum("bmhd,blhd->bhml", q, k) / jnp.sqrt(float(dh))
    attn = jax.nn.softmax(logits, axis=-1)
    ctx = jnp.einsum("bhml,blhd->bmhd", attn, v).reshape(B, M, D)
    s = ctx @ w_s
    rho2, r2, nf2, aux = _rho_pallas(s, a_r, b_r)
    rho = rho2[:, 0]
    r = r2[:, 0]
    n_vec = nf2[:, 0].astype(jnp.int32)
    _, top_idx = jax.lax.top_k(s, NMAX)
    rows = _sc_gather(X_v.reshape(B * M, D), top_idx.reshape(NTOT))
    Z = _z_pallas(rows, W_re).reshape(B, NMAX, D)
    return (
        Z,
        top_idx,
        rho,
        r,
        n_vec,
        aux[0, 0],
        aux[1, 0],
        aux[2, 0],
    )
